# baseline (speedup 1.0000x reference)
import jax
import jax.numpy as jnp
from jax import lax
from jax.experimental import pallas as pl
from jax.experimental.pallas import tpu as pltpu
from jax.experimental.pallas import tpu_sc as plsc

_H = 224
_W = 224
_K = 16
_LH = 14
_LW = 14
_KK = _K * _K
_L = _LH * _LW


def _fold_body(in_hbm, out_hbm, in_buf, out_buf):
    info = plsc.get_sparse_core_info()
    nc, ns = info.num_cores, info.num_subcores
    nw = nc * ns
    wid = lax.axis_index("s") * nc + lax.axis_index("c")
    n_slabs = out_hbm.shape[0] * out_hbm.shape[1]
    n_c = out_hbm.shape[1]
    per_w = n_slabs // nw
    lanes = lax.iota(jnp.int32, 16)
    p_vecs = [lanes + kh * _K for kh in range(_K)]

    def do_slab(i, carry):
        slab = wid * per_w + i
        b = slab // n_c
        c = slab - b * n_c
        pltpu.sync_copy(in_hbm.at[b, pl.ds(c * _KK, _KK), :], in_buf)

        @plsc.parallel_loop(0, _LH)
        def row(oh):
            l_oh = oh * _LW
            for kh in range(_K):
                t = oh * _K + kh
                for ow in range(_LW):
                    l_vec = jnp.full((16,), l_oh + ow, jnp.int32)
                    out_buf[t, pl.ds(ow * _K, _K)] = plsc.load_gather(
                        in_buf, [p_vecs[kh], l_vec]
                    )

        pltpu.sync_copy(out_buf, out_hbm.at[b, c])
        return carry

    lax.fori_loop(0, per_w, do_slab, 0)


def kernel(input):
    B, CK, _ = input.shape
    C = CK // _KK
    mesh = plsc.VectorSubcoreMesh(core_axis_name="c", subcore_axis_name="s")
    return pl.kernel(
        _fold_body,
        out_type=jax.ShapeDtypeStruct((B, C, _H, _W), jnp.float32),
        mesh=mesh,
        scratch_types=[
            pltpu.VMEM((_KK, _L), jnp.float32),
            pltpu.VMEM((_H, _W), jnp.float32),
        ],
        compiler_params=pltpu.CompilerParams(needs_layout_passes=False),
    )(input)
